# Initial kernel scaffold; baseline (speedup 1.0000x reference)
import jax
import jax.numpy as jnp
from jax.experimental import pallas as pl


def kernel(x, edge_index, W_l1, b1, W_r1, gamma1, beta1, rm1, rv1, W_l2, b2, W_r2):
    raise NotImplementedError("write your pallas kernel here")


# speedup vs baseline: 6.2606x; 6.2606x over previous
import functools

import jax
import jax.numpy as jnp
from jax import lax
from jax.experimental import pallas as pl
from jax.experimental.pallas import tpu as pltpu
from jax.experimental.pallas import tpu_sc as plsc

N = 10000
E = 320000
D_IN = 128
D_H = 256
D_OUT = 128

NC = 2
NS = 16
NW = NC * NS
CH = 80
EPW = E // NW
NCHUNK = EPW // CH
NP = 10240
RPT = NP // NS

_mesh = plsc.VectorSubcoreMesh(core_axis_name="c", subcore_axis_name="s")


def _seg_sum_body(with_cnt, feat_hbm, src_hbm, dst_hbm, zf_hbm, zc_hbm,
                  ones_hbm, acc_out, cnt_out, src_v, dst_v, rows_v,
                  ones_v, acc_sh, cnt_sh, sem):
    cid = lax.axis_index("c")
    sid = lax.axis_index("s")
    wid = cid * NS + sid
    r0 = sid * RPT

    pltpu.sync_copy(zf_hbm.at[pl.ds(r0, RPT)], acc_sh.at[pl.ds(r0, RPT)])
    if with_cnt:
        pltpu.sync_copy(zc_hbm.at[pl.ds(r0, RPT)], cnt_sh.at[pl.ds(r0, RPT)])
        pltpu.sync_copy(ones_hbm, ones_v)
    plsc.subcore_barrier()
    base_e = wid * EPW

    def body(j, carry):
        off = base_e + j * CH
        pltpu.sync_copy(src_hbm.at[pl.ds(off, CH)], src_v)
        pltpu.sync_copy(dst_hbm.at[pl.ds(off, CH)], dst_v)
        pltpu.async_copy(feat_hbm.at[src_v], rows_v, sem).wait()
        pltpu.sync_copy(rows_v, acc_sh.at[dst_v], add=True)
        if with_cnt:
            pltpu.sync_copy(ones_v, cnt_sh.at[dst_v], add=True)
        return carry

    lax.fori_loop(0, NCHUNK, body, 0)
    plsc.subcore_barrier()

    out_r0 = cid * NP + r0
    pltpu.sync_copy(acc_sh.at[pl.ds(r0, RPT)], acc_out.at[pl.ds(out_r0, RPT)])
    if with_cnt:
        pltpu.sync_copy(cnt_sh.at[pl.ds(r0, RPT)],
                        cnt_out.at[pl.ds(out_r0, RPT)])


@functools.partial(
    pl.kernel,
    mesh=_mesh,
    out_type=[
        jax.ShapeDtypeStruct((NC * NP, D_IN), jnp.float32),
        jax.ShapeDtypeStruct((NC * NP,), jnp.float32),
    ],
    scratch_types=[
        pltpu.VMEM((CH,), jnp.int32),
        pltpu.VMEM((CH,), jnp.int32),
        pltpu.VMEM((CH, D_IN), jnp.float32),
        pltpu.VMEM((CH,), jnp.float32),
        pltpu.VMEM_SHARED((NP, D_IN), jnp.float32),
        pltpu.VMEM_SHARED((NP,), jnp.float32),
        pltpu.SemaphoreType.DMA,
    ],
)
def _seg_sum_cnt_kernel(feat_hbm, src_hbm, dst_hbm, zf_hbm, zc_hbm, ones_hbm,
                        acc_out, cnt_out, src_v, dst_v, rows_v, ones_v,
                        acc_sh, cnt_sh, sem):
    _seg_sum_body(True, feat_hbm, src_hbm, dst_hbm, zf_hbm, zc_hbm, ones_hbm,
                  acc_out, cnt_out, src_v, dst_v, rows_v, ones_v, acc_sh,
                  cnt_sh, sem)


@functools.partial(
    pl.kernel,
    mesh=_mesh,
    out_type=jax.ShapeDtypeStruct((NC * NP, D_IN), jnp.float32),
    scratch_types=[
        pltpu.VMEM((CH,), jnp.int32),
        pltpu.VMEM((CH,), jnp.int32),
        pltpu.VMEM((CH, D_IN), jnp.float32),
        pltpu.VMEM_SHARED((NP, D_IN), jnp.float32),
        pltpu.SemaphoreType.DMA,
    ],
)
def _seg_sum_kernel(feat_hbm, src_hbm, dst_hbm, zf_hbm, acc_out, src_v,
                    dst_v, rows_v, acc_sh, sem):
    _seg_sum_body(False, feat_hbm, src_hbm, dst_hbm, zf_hbm, None, None,
                  acc_out, None, src_v, dst_v, rows_v, None, acc_sh,
                  None, sem)


def _tc1_body(x_ref, a0_ref, a1_ref, c0_ref, c1_ref, wl1_ref, wr1_ref,
              b1_ref, g_ref, be_ref, rm_ref, rv_ref, wl2_ref, h_ref, y_ref):
    cnt = c0_ref[...] + c1_ref[...]
    mean = (a0_ref[...] + a1_ref[...]) / jnp.maximum(cnt, 1.0)
    z = (jnp.dot(mean, wl1_ref[...], preferred_element_type=jnp.float32)
         + jnp.dot(x_ref[...], wr1_ref[...], preferred_element_type=jnp.float32)
         + b1_ref[...])
    scale = g_ref[...] * lax.rsqrt(rv_ref[...] + 1e-5)
    h = jnp.maximum((z - rm_ref[...]) * scale + be_ref[...], 0.0)
    h_ref[...] = h
    y_ref[...] = jnp.dot(h, wl2_ref[...], preferred_element_type=jnp.float32)


def _tc2_body(h_ref, a0_ref, a1_ref, c0_ref, c1_ref, wr2_ref, b2_ref,
              out_ref):
    cnt = c0_ref[...] + c1_ref[...]
    mean = (a0_ref[...] + a1_ref[...]) / jnp.maximum(cnt, 1.0)
    out_ref[...] = (
        mean + b2_ref[...]
        + jnp.dot(h_ref[...], wr2_ref[...], preferred_element_type=jnp.float32))


BR = 1000


def _row_spec(d):
    return pl.BlockSpec((BR, d), lambda i: (i, 0))


def _full_spec(r, d):
    return pl.BlockSpec((r, d), lambda i: (0, 0))


_tc1 = pl.pallas_call(
    _tc1_body,
    grid=(N // BR,),
    in_specs=[
        _row_spec(D_IN), _row_spec(D_IN), _row_spec(D_IN),
        _row_spec(1), _row_spec(1),
        _full_spec(D_IN, D_H), _full_spec(D_IN, D_H), _full_spec(1, D_H),
        _full_spec(1, D_H), _full_spec(1, D_H), _full_spec(1, D_H),
        _full_spec(1, D_H), _full_spec(D_H, D_OUT),
    ],
    out_specs=[_row_spec(D_H), _row_spec(D_OUT)],
    out_shape=[
        jax.ShapeDtypeStruct((N, D_H), jnp.float32),
        jax.ShapeDtypeStruct((N, D_OUT), jnp.float32),
    ],
)

_tc2 = pl.pallas_call(
    _tc2_body,
    grid=(N // BR,),
    in_specs=[
        _row_spec(D_H), _row_spec(D_OUT), _row_spec(D_OUT),
        _row_spec(1), _row_spec(1),
        _full_spec(D_H, D_OUT), _full_spec(1, D_OUT),
    ],
    out_specs=_row_spec(D_OUT),
    out_shape=jax.ShapeDtypeStruct((N, D_OUT), jnp.float32),
)


def kernel(x, edge_index, W_l1, b1, W_r1, gamma1, beta1, rm1, rv1,
           W_l2, b2, W_r2):
    src = edge_index[0].astype(jnp.int32)
    dst = edge_index[1].astype(jnp.int32)
    zf = jnp.zeros((NP, D_IN), jnp.float32)
    zc = jnp.zeros((NP,), jnp.float32)
    ones = jnp.ones((CH,), jnp.float32)

    acc1, cnt1 = _seg_sum_cnt_kernel(x, src, dst, zf, zc, ones)
    c0 = cnt1[:N].reshape(N, 1)
    c1 = cnt1[NP:NP + N].reshape(N, 1)
    h, y = _tc1(
        x, acc1[:N], acc1[NP:NP + N], c0, c1,
        W_l1, W_r1, b1.reshape(1, D_H), gamma1.reshape(1, D_H),
        beta1.reshape(1, D_H), rm1.reshape(1, D_H), rv1.reshape(1, D_H),
        W_l2)
    acc2 = _seg_sum_kernel(y, src, dst, zf)
    out = _tc2(h, acc2[:N], acc2[NP:NP + N], c0, c1,
               W_r2, b2.reshape(1, D_OUT))
    return out
